# baseline (speedup 1.0000x reference)
import functools

import jax
import jax.numpy as jnp
from jax import lax
from jax.experimental import pallas as pl
from jax.experimental.pallas import tpu as pltpu
from jax.experimental.pallas import tpu_sc as plsc

_N = 10000
_E = 320000
_D = 128
_DE = 16
_H = 8
_DH = _D // _H
_FFN = 512

_BN = 512
_BE = 4096
_CA = 1600
_CC = 800
_EQ = _E // 4

_f32 = jnp.float32
_i32 = jnp.int32

_SC_PARAMS = pltpu.CompilerParams(use_tc_tiling_on_sc=False,
                                  needs_layout_passes=False)


def _rne_bf16_bits(v):
    iv = jax.lax.bitcast_convert_type(v, _i32)
    return ((iv + 0x7FFF + ((iv >> 16) & 1)) >> 16) & 0xFFFF


def _node_pre_body(x_ref, w_ref, s_ref, hsp_ref):
    p = jnp.dot(x_ref[...], w_ref[...], preferred_element_type=_f32)
    s_ref[...] = p[:, _D:].T
    lo = _rne_bf16_bits(p[:, :_D // 2])
    hi = _rne_bf16_bits(p[:, _D // 2:_D])
    hsp_ref[...] = (lo | (hi << 16)).T


def _node_pre(x, w_cat):
    k = w_cat.shape[1]
    return pl.pallas_call(
        _node_pre_body,
        grid=(pl.cdiv(_N, _BN),),
        in_specs=[
            pl.BlockSpec((_BN, _D), lambda i: (i, 0)),
            pl.BlockSpec((_D, k), lambda i: (0, 0)),
        ],
        out_specs=[
            pl.BlockSpec((2 * _H, _BN), lambda i: (0, i)),
            pl.BlockSpec((_D // 2, _BN), lambda i: (0, i)),
        ],
        out_shape=[
            jax.ShapeDtypeStruct((2 * _H, _N), _f32),
            jax.ShapeDtypeStruct((_D // 2, _N), _i32),
        ],
    )(x, w_cat)


def _edge_pre_body(eat_ref, vt_ref, se_ref, eap_ref):
    ea = eat_ref[...]
    se_ref[...] = jnp.dot(vt_ref[...], ea,
                          preferred_element_type=_f32)
    lo = _rne_bf16_bits(ea[:_DE // 2])
    hi = _rne_bf16_bits(ea[_DE // 2:])
    eap_ref[...] = lo | (hi << 16)


def _edge_pre(eaT, v_edgeT):
    return pl.pallas_call(
        _edge_pre_body,
        grid=(pl.cdiv(_E, _BE),),
        in_specs=[
            pl.BlockSpec((_DE, _BE), lambda i: (0, i)),
            pl.BlockSpec((_H, _DE), lambda i: (0, 0)),
        ],
        out_specs=[
            pl.BlockSpec((_H, _BE), lambda i: (0, i)),
            pl.BlockSpec((_DE // 2, _BE), lambda i: (0, i)),
        ],
        out_shape=[
            jax.ShapeDtypeStruct((_H, _E), _f32),
            jax.ShapeDtypeStruct((_DE // 2, _E), _i32),
        ],
    )(eaT, v_edgeT)


def _denred_body(p_ref, o_ref):
    p = p_ref[...].reshape(2, 4, 4, _N).sum(axis=1).reshape(_H, _N)
    o_ref[...] = 1.0 / (p + 1e-9)


def _denred(partials):
    return pl.pallas_call(
        _denred_body,
        out_shape=jax.ShapeDtypeStruct((_H, _N), _f32),
    )(partials)


def _ffn_body(x_ref, aggt_ref, bt_ref, wbd_ref, w1_ref, b1_ref, w2_ref,
              b2_ref, o_ref):
    a = aggt_ref[...].T + jnp.dot(bt_ref[...].T, wbd_ref[...],
                                  preferred_element_type=_f32)
    elu = jnp.where(a > 0, a, jnp.exp(jnp.minimum(a, 0.0)) - 1.0)
    h1 = x_ref[...] + elu
    t = jnp.maximum(
        jnp.dot(h1, w1_ref[...], preferred_element_type=_f32)
        + b1_ref[...], 0.0)
    o_ref[...] = h1 + jnp.dot(t, w2_ref[...],
                              preferred_element_type=_f32) + b2_ref[...]


def _ffn(x, aggT, bT, w_bd, w1, b1, w2, b2):
    return pl.pallas_call(
        _ffn_body,
        grid=(pl.cdiv(_N, _BN),),
        in_specs=[
            pl.BlockSpec((_BN, _D), lambda i: (i, 0)),
            pl.BlockSpec((_D, _BN), lambda i: (0, i)),
            pl.BlockSpec((_D, _BN), lambda i: (0, i)),
            pl.BlockSpec((_D, _D), lambda i: (0, 0)),
            pl.BlockSpec((_D, _FFN), lambda i: (0, 0)),
            pl.BlockSpec((1, _FFN), lambda i: (0, 0)),
            pl.BlockSpec((_FFN, _D), lambda i: (0, 0)),
            pl.BlockSpec((1, _D), lambda i: (0, 0)),
        ],
        out_specs=pl.BlockSpec((_BN, _D), lambda i: (i, 0)),
        out_shape=jax.ShapeDtypeStruct((_N, _D), _f32),
    )(x, aggT, bT, w_bd, w1, b1.reshape(1, _FFN), w2, b2.reshape(1, _D))


_MESH = plsc.VectorSubcoreMesh(core_axis_name="c", subcore_axis_name="s",
                               num_cores=2, num_subcores=16)


@functools.partial(
    pl.kernel,
    out_type=(
        jax.ShapeDtypeStruct((_H, _E), _f32),
        jax.ShapeDtypeStruct((2, 16, _N), _f32),
    ),
    mesh=_MESH,
    compiler_params=_SC_PARAMS,
    scratch_types=[
        pltpu.VMEM((_N,), _f32),
        pltpu.VMEM((_N,), _f32),
        pltpu.VMEM((_N,), _f32),
        pltpu.VMEM((_CA,), _i32),
        pltpu.VMEM((_CA,), _i32),
        pltpu.VMEM((_CA,), _f32),
        pltpu.VMEM((_CA,), _i32),
        pltpu.VMEM((_CA,), _i32),
        pltpu.VMEM((_CA,), _f32),
        pltpu.VMEM((_CA,), _f32),
        pltpu.SemaphoreType.DMA,
        pltpu.SemaphoreType.DMA,
    ],
)
def _sc_pass_a(nodeS, seT, src_e, dst_e, exT_out, den_out,
               ssrc_v, sdst_v, den_v,
               srcA0, dstA0, seA0, srcA1, dstA1, seA1, ex_v, semA0, semA1):
    c = lax.axis_index("c")
    s = lax.axis_index("s")
    h = 4 * c + lax.rem(s, 4)
    q = lax.div(s, 4)

    pltpu.sync_copy(nodeS.at[h], ssrc_v)
    pltpu.sync_copy(nodeS.at[_H + h], sdst_v)

    def zero(i, carry):
        den_v[pl.ds(i * 16, 16)] = jnp.zeros((16,), _f32)
        return carry
    lax.fori_loop(0, _N // 16, zero, 0)

    bufs = [(srcA0, dstA0, seA0, semA0), (srcA1, dstA1, seA1, semA1)]
    nch = _EQ // _CA

    def start(k, b):
        sv, dv, sev, sem = bufs[b]
        e0 = q * _EQ + k * _CA
        pltpu.async_copy(src_e.at[pl.ds(e0, _CA)], sv, sem)
        pltpu.async_copy(dst_e.at[pl.ds(e0, _CA)], dv, sem)
        pltpu.async_copy(seT.at[h, pl.ds(e0, _CA)], sev, sem)

    def wait(b):
        sv, dv, sev, sem = bufs[b]
        pltpu.make_async_copy(src_e.at[pl.ds(0, _CA)], sv, sem).wait()
        pltpu.make_async_copy(dst_e.at[pl.ds(0, _CA)], dv, sem).wait()
        pltpu.make_async_copy(seT.at[0, pl.ds(0, _CA)], sev, sem).wait()

    _GA = 4

    def process(b, k):
        sv, dv, sev, _sem = bufs[b]

        def grp(g, carry2):
            srcs, dsts, ses, gs, gd = [], [], [], [], []
            for u in range(_GA):
                sl = pl.ds((g * _GA + u) * 16, 16)
                srcv = sv[sl]
                dstv = dv[sl]
                srcs.append(srcv)
                dsts.append(dstv)
                ses.append(sev[sl])
                gs.append(plsc.load_gather(ssrc_v, [srcv]))
                gd.append(plsc.load_gather(sdst_v, [dstv]))
            for u in range(_GA):
                sl = pl.ds((g * _GA + u) * 16, 16)
                sc = gs[u] + gd[u] + ses[u]
                sc = jnp.maximum(sc, 0.2 * sc)
                ex = jnp.exp(sc)
                ex_v[sl] = ex
                plsc.addupdate_scatter(den_v, [dsts[u]], ex)
            return carry2
        lax.fori_loop(0, _CA // (16 * _GA), grp, 0)
        e0 = q * _EQ + k * _CA
        pltpu.sync_copy(ex_v, exT_out.at[h, pl.ds(e0, _CA)])

    start(0, 0)
    start(1, 1)

    def body(i, carry):
        for b in (0, 1):
            k = 2 * i + b
            wait(b)
            process(b, k)

            @pl.when(k + 2 < nch)
            def _():
                start(k + 2, b)
        return carry
    lax.fori_loop(0, nch // 2, body, 0)

    pltpu.sync_copy(den_v, den_out.at[c, s])


@functools.partial(
    pl.kernel,
    out_type=(
        jax.ShapeDtypeStruct((_D, _N), _f32),
        jax.ShapeDtypeStruct((_D, _N), _f32),
    ),
    mesh=_MESH,
    compiler_params=_SC_PARAMS,
    scratch_types=[
        pltpu.VMEM((2, _N), _i32),
        pltpu.VMEM((4, _N), _f32),
        pltpu.VMEM((4, _N), _f32),
        pltpu.VMEM((_N,), _f32),
        pltpu.VMEM((_CC,), _i32),
        pltpu.VMEM((_CC,), _i32),
        pltpu.VMEM((_CC,), _f32),
        pltpu.VMEM((2, _CC), _i32),
        pltpu.VMEM((_CC,), _i32),
        pltpu.VMEM((_CC,), _i32),
        pltpu.VMEM((_CC,), _f32),
        pltpu.VMEM((2, _CC), _i32),
        pltpu.SemaphoreType.DMA,
        pltpu.SemaphoreType.DMA,
    ],
)
def _sc_pass_c(hsP, eaP, src_e, dst_e, rdenT, exT_in, aggT_out, bT_out,
               hs_v, agg_v, b_v, rden_v,
               src0, dst0, ex0, ea0, src1, dst1, ex1, ea1, sem0, sem1):
    c = lax.axis_index("c")
    s = lax.axis_index("s")
    f0 = 64 * c + 4 * s
    h = lax.div(16 * c + s, 4)
    k0 = 4 * lax.rem(s, 4)
    bq0 = 16 * h + k0

    pltpu.sync_copy(hsP.at[pl.ds(lax.div(f0, 2), 2)], hs_v)
    pltpu.sync_copy(rdenT.at[h], rden_v)

    def zero(i, carry):
        z = jnp.zeros((16,), _f32)
        agg_v[0, pl.ds(i * 16, 16)] = z
        agg_v[1, pl.ds(i * 16, 16)] = z
        agg_v[2, pl.ds(i * 16, 16)] = z
        agg_v[3, pl.ds(i * 16, 16)] = z
        b_v[0, pl.ds(i * 16, 16)] = z
        b_v[1, pl.ds(i * 16, 16)] = z
        b_v[2, pl.ds(i * 16, 16)] = z
        b_v[3, pl.ds(i * 16, 16)] = z
        return carry
    lax.fori_loop(0, _N // 16, zero, 0)

    jv = [jnp.full((16,), j, _i32) for j in range(4)]
    jw = [jnp.full((16,), j, _i32) for j in range(2)]
    bufs = [(src0, dst0, ex0, ea0, sem0), (src1, dst1, ex1, ea1, sem1)]
    nch = _E // _CC

    def start(k, b):
        sv, dv, ev, eav, sem = bufs[b]
        e0 = k * _CC
        pltpu.async_copy(src_e.at[pl.ds(e0, _CC)], sv, sem)
        pltpu.async_copy(dst_e.at[pl.ds(e0, _CC)], dv, sem)
        pltpu.async_copy(exT_in.at[h, pl.ds(e0, _CC)], ev, sem)
        pltpu.async_copy(eaP.at[pl.ds(lax.div(k0, 2), 2), pl.ds(e0, _CC)],
                         eav, sem)

    def wait(b):
        sv, dv, ev, eav, sem = bufs[b]
        pltpu.make_async_copy(src_e.at[pl.ds(0, _CC)], sv, sem).wait()
        pltpu.make_async_copy(dst_e.at[pl.ds(0, _CC)], dv, sem).wait()
        pltpu.make_async_copy(exT_in.at[0, pl.ds(0, _CC)], ev, sem).wait()
        pltpu.make_async_copy(eaP.at[pl.ds(0, 2), pl.ds(0, _CC)], eav,
                              sem).wait()

    _GC = 5

    def process(b):
        sv, dv, ev, eav, _sem = bufs[b]

        def grp(g, carry2):
            dsts, exs, hsp, eas = [], [], [], []
            for u in range(_GC):
                sl = pl.ds((g * _GC + u) * 16, 16)
                srcv = sv[sl]
                dsts.append(dv[sl])
                exs.append(ev[sl])
                hsp.append([plsc.load_gather(hs_v, [jw[w], srcv])
                            for w in range(2)])
                eas.append([eav[w, sl] for w in range(2)])
            for u in range(_GC):
                exv = exs[u]
                for w in range(2):
                    g16 = hsp[u][w]
                    lo = plsc.bitcast(jnp.left_shift(g16, 16), _f32)
                    hi = plsc.bitcast(
                        jnp.bitwise_and(g16, jnp.int32(-65536)), _f32)
                    plsc.addupdate_scatter(agg_v, [jv[2 * w], dsts[u]],
                                           lo * exv)
                    plsc.addupdate_scatter(agg_v, [jv[2 * w + 1], dsts[u]],
                                           hi * exv)
                    e16 = eas[u][w]
                    elo = plsc.bitcast(jnp.left_shift(e16, 16), _f32)
                    ehi = plsc.bitcast(
                        jnp.bitwise_and(e16, jnp.int32(-65536)), _f32)
                    plsc.addupdate_scatter(b_v, [jv[2 * w], dsts[u]],
                                           elo * exv)
                    plsc.addupdate_scatter(b_v, [jv[2 * w + 1], dsts[u]],
                                           ehi * exv)
            return carry2
        lax.fori_loop(0, _CC // (16 * _GC), grp, 0)

    start(0, 0)
    start(1, 1)

    def body(i, carry):
        for b in (0, 1):
            k = 2 * i + b
            wait(b)
            process(b)

            @pl.when(k + 2 < nch)
            def _():
                start(k + 2, b)
        return carry
    lax.fori_loop(0, nch // 2, body, 0)

    def scale(i, carry):
        sl = pl.ds(i * 16, 16)
        rv = rden_v[sl]
        for r in range(4):
            agg_v[r, sl] = agg_v[r, sl] * rv
            b_v[r, sl] = b_v[r, sl] * rv
        return carry
    lax.fori_loop(0, _N // 16, scale, 0)

    pltpu.sync_copy(agg_v, aggT_out.at[pl.ds(f0, 4)])
    pltpu.sync_copy(b_v, bT_out.at[pl.ds(bq0, 4)])


def kernel(x, edge_index, edge_attr, W_src, W_dst, W_edge, a_src, a_dst,
           a_edge, W1, b1, W2, b2):
    v_src = (W_src.reshape(_D, _H, _DH) * a_src[None]).sum(-1)
    v_dst = (W_dst.reshape(_D, _H, _DH) * a_dst[None]).sum(-1)
    v_edge = (W_edge.reshape(_DE, _H, _DH) * a_edge[None]).sum(-1)
    w_src_p = jnp.concatenate([W_src[:, 0::2], W_src[:, 1::2]], axis=1)
    w_node = jnp.concatenate([w_src_p, v_src, v_dst], axis=1)
    w_bd = jax.scipy.linalg.block_diag(
        *[W_edge[:, 16 * hh:16 * hh + 16] for hh in range(_H)])

    nodeS, hsP = _node_pre(x, w_node)
    kperm = [2 * i for i in range(_DE // 2)] + [2 * i + 1 for i in range(_DE // 2)]
    eaTp = edge_attr[:, jnp.array(kperm)].T
    seT, eaP = _edge_pre(eaTp, v_edge[jnp.array(kperm), :].T)

    src_e = edge_index[0]
    dst_e = edge_index[1]
    exT, den_part = _sc_pass_a(nodeS, seT, src_e, dst_e)
    rdenT = _denred(den_part)
    aggT, bT = _sc_pass_c(hsP, eaP, src_e, dst_e, rdenT, exT)

    return _ffn(x, aggT, bT, w_bd, W1, b1, W2, b2)


# speedup vs baseline: 1.1022x; 1.1022x over previous
import functools

import jax
import jax.numpy as jnp
from jax import lax
from jax.experimental import pallas as pl
from jax.experimental.pallas import tpu as pltpu
from jax.experimental.pallas import tpu_sc as plsc

_N = 10000
_E = 320000
_D = 128
_DE = 16
_H = 8
_DH = _D // _H
_FFN = 512

_BN = 512
_BE = 4096
_CA = 1600
_CC = 800
_EQ = _E // 4

_f32 = jnp.float32
_i32 = jnp.int32

_SC_PARAMS = pltpu.CompilerParams(use_tc_tiling_on_sc=False,
                                  needs_layout_passes=False)


def _rne_bf16_bits(v):
    iv = jax.lax.bitcast_convert_type(v, _i32)
    return ((iv + 0x7FFF + ((iv >> 16) & 1)) >> 16) & 0xFFFF


def _node_pre_body(x_ref, w_ref, s_ref, hsp_ref):
    p = jnp.dot(x_ref[...], w_ref[...], preferred_element_type=_f32)
    s_ref[...] = p[:, _D:].T
    lo = _rne_bf16_bits(p[:, :_D // 2])
    hi = _rne_bf16_bits(p[:, _D // 2:_D])
    hsp_ref[...] = (lo | (hi << 16)).T


def _node_pre(x, w_cat):
    k = w_cat.shape[1]
    return pl.pallas_call(
        _node_pre_body,
        grid=(pl.cdiv(_N, _BN),),
        in_specs=[
            pl.BlockSpec((_BN, _D), lambda i: (i, 0)),
            pl.BlockSpec((_D, k), lambda i: (0, 0)),
        ],
        out_specs=[
            pl.BlockSpec((2 * _H, _BN), lambda i: (0, i)),
            pl.BlockSpec((_D // 2, _BN), lambda i: (0, i)),
        ],
        out_shape=[
            jax.ShapeDtypeStruct((2 * _H, _N), _f32),
            jax.ShapeDtypeStruct((_D // 2, _N), _i32),
        ],
    )(x, w_cat)


def _edge_pre_body(eat_ref, vt_ref, se_ref, eap_ref):
    ea = eat_ref[...]
    se_ref[...] = jnp.dot(vt_ref[...], ea,
                          preferred_element_type=_f32)
    lo = _rne_bf16_bits(ea[:_DE // 2])
    hi = _rne_bf16_bits(ea[_DE // 2:])
    eap_ref[...] = lo | (hi << 16)


def _edge_pre(eaT, v_edgeT):
    return pl.pallas_call(
        _edge_pre_body,
        grid=(pl.cdiv(_E, _BE),),
        in_specs=[
            pl.BlockSpec((_DE, _BE), lambda i: (0, i)),
            pl.BlockSpec((_H, _DE), lambda i: (0, 0)),
        ],
        out_specs=[
            pl.BlockSpec((_H, _BE), lambda i: (0, i)),
            pl.BlockSpec((_DE // 2, _BE), lambda i: (0, i)),
        ],
        out_shape=[
            jax.ShapeDtypeStruct((_H, _E), _f32),
            jax.ShapeDtypeStruct((_DE // 2, _E), _i32),
        ],
    )(eaT, v_edgeT)


def _denred_body(p_ref, o_ref):
    p = p_ref[...].reshape(2, 4, 4, _N).sum(axis=1).reshape(_H, _N)
    o_ref[...] = 1.0 / (p + 1e-9)


def _denred(partials):
    return pl.pallas_call(
        _denred_body,
        out_shape=jax.ShapeDtypeStruct((_H, _N), _f32),
    )(partials)


def _ffn_body(x_ref, aggt_ref, bt_ref, wbd_ref, w1_ref, b1_ref, w2_ref,
              b2_ref, o_ref):
    a = aggt_ref[...].T + jnp.dot(bt_ref[...].T, wbd_ref[...],
                                  preferred_element_type=_f32)
    elu = jnp.where(a > 0, a, jnp.exp(jnp.minimum(a, 0.0)) - 1.0)
    h1 = x_ref[...] + elu
    t = jnp.maximum(
        jnp.dot(h1, w1_ref[...], preferred_element_type=_f32)
        + b1_ref[...], 0.0)
    o_ref[...] = h1 + jnp.dot(t, w2_ref[...],
                              preferred_element_type=_f32) + b2_ref[...]


def _ffn(x, aggT, bT, w_bd, w1, b1, w2, b2):
    return pl.pallas_call(
        _ffn_body,
        grid=(pl.cdiv(_N, _BN),),
        in_specs=[
            pl.BlockSpec((_BN, _D), lambda i: (i, 0)),
            pl.BlockSpec((_D, _BN), lambda i: (0, i)),
            pl.BlockSpec((_D, _BN), lambda i: (0, i)),
            pl.BlockSpec((_D, _D), lambda i: (0, 0)),
            pl.BlockSpec((_D, _FFN), lambda i: (0, 0)),
            pl.BlockSpec((1, _FFN), lambda i: (0, 0)),
            pl.BlockSpec((_FFN, _D), lambda i: (0, 0)),
            pl.BlockSpec((1, _D), lambda i: (0, 0)),
        ],
        out_specs=pl.BlockSpec((_BN, _D), lambda i: (i, 0)),
        out_shape=jax.ShapeDtypeStruct((_N, _D), _f32),
    )(x, aggT, bT, w_bd, w1, b1.reshape(1, _FFN), w2, b2.reshape(1, _D))


_MESH = plsc.VectorSubcoreMesh(core_axis_name="c", subcore_axis_name="s",
                               num_cores=2, num_subcores=16)


@functools.partial(
    pl.kernel,
    out_type=(
        jax.ShapeDtypeStruct((_H, _E), _f32),
        jax.ShapeDtypeStruct((2, 16, _N), _f32),
    ),
    mesh=_MESH,
    compiler_params=_SC_PARAMS,
    scratch_types=[
        pltpu.VMEM((_N,), _f32),
        pltpu.VMEM((_N,), _f32),
        pltpu.VMEM((_N,), _f32),
        pltpu.VMEM((_CA,), _i32),
        pltpu.VMEM((_CA,), _i32),
        pltpu.VMEM((_CA,), _f32),
        pltpu.VMEM((_CA,), _i32),
        pltpu.VMEM((_CA,), _i32),
        pltpu.VMEM((_CA,), _f32),
        pltpu.VMEM((_CA,), _f32),
        pltpu.SemaphoreType.DMA,
        pltpu.SemaphoreType.DMA,
    ],
)
def _sc_pass_a(nodeS, seT, src_e, dst_e, exT_out, den_out,
               ssrc_v, sdst_v, den_v,
               srcA0, dstA0, seA0, srcA1, dstA1, seA1, ex_v, semA0, semA1):
    c = lax.axis_index("c")
    s = lax.axis_index("s")
    h = 4 * c + lax.rem(s, 4)
    q = lax.div(s, 4)

    pltpu.sync_copy(nodeS.at[h], ssrc_v)
    pltpu.sync_copy(nodeS.at[_H + h], sdst_v)

    def zero(i, carry):
        den_v[pl.ds(i * 16, 16)] = jnp.zeros((16,), _f32)
        return carry
    lax.fori_loop(0, _N // 16, zero, 0)

    bufs = [(srcA0, dstA0, seA0, semA0), (srcA1, dstA1, seA1, semA1)]
    nch = _EQ // _CA

    def start(k, b):
        sv, dv, sev, sem = bufs[b]
        e0 = q * _EQ + k * _CA
        pltpu.async_copy(src_e.at[pl.ds(e0, _CA)], sv, sem)
        pltpu.async_copy(dst_e.at[pl.ds(e0, _CA)], dv, sem)
        pltpu.async_copy(seT.at[h, pl.ds(e0, _CA)], sev, sem)

    def wait(b):
        sv, dv, sev, sem = bufs[b]
        pltpu.make_async_copy(src_e.at[pl.ds(0, _CA)], sv, sem).wait()
        pltpu.make_async_copy(dst_e.at[pl.ds(0, _CA)], dv, sem).wait()
        pltpu.make_async_copy(seT.at[0, pl.ds(0, _CA)], sev, sem).wait()

    _GA = 4

    def process(b, k):
        sv, dv, sev, _sem = bufs[b]

        def grp(g, carry2):
            srcs, dsts, ses, gs, gd = [], [], [], [], []
            for u in range(_GA):
                sl = pl.ds((g * _GA + u) * 16, 16)
                srcv = sv[sl]
                dstv = dv[sl]
                srcs.append(srcv)
                dsts.append(dstv)
                ses.append(sev[sl])
                gs.append(plsc.load_gather(ssrc_v, [srcv]))
                gd.append(plsc.load_gather(sdst_v, [dstv]))
            for u in range(_GA):
                sl = pl.ds((g * _GA + u) * 16, 16)
                sc = gs[u] + gd[u] + ses[u]
                sc = jnp.maximum(sc, 0.2 * sc)
                ex = jnp.exp(sc)
                ex_v[sl] = ex
                plsc.addupdate_scatter(den_v, [dsts[u]], ex)
            return carry2
        lax.fori_loop(0, _CA // (16 * _GA), grp, 0)
        e0 = q * _EQ + k * _CA
        pltpu.sync_copy(ex_v, exT_out.at[h, pl.ds(e0, _CA)])

    start(0, 0)
    start(1, 1)

    def body(i, carry):
        for b in (0, 1):
            k = 2 * i + b
            wait(b)
            process(b, k)

            @pl.when(k + 2 < nch)
            def _():
                start(k + 2, b)
        return carry
    lax.fori_loop(0, nch // 2, body, 0)

    pltpu.sync_copy(den_v, den_out.at[c, s])


@functools.partial(
    pl.kernel,
    out_type=(
        jax.ShapeDtypeStruct((_D, _N), _f32),
        jax.ShapeDtypeStruct((_D, _N), _f32),
    ),
    mesh=_MESH,
    compiler_params=_SC_PARAMS,
    scratch_types=[
        pltpu.VMEM((2, _N), _i32),
        pltpu.VMEM((4, _N), _f32),
        pltpu.VMEM((2, _N), _f32),
        pltpu.VMEM((2, _N), _f32),
        pltpu.VMEM((_N,), _f32),
        pltpu.VMEM((_CC,), _i32),
        pltpu.VMEM((_CC,), _i32),
        pltpu.VMEM((_CC,), _f32),
        pltpu.VMEM((2, _CC), _i32),
        pltpu.VMEM((_CC,), _i32),
        pltpu.VMEM((_CC,), _i32),
        pltpu.VMEM((_CC,), _f32),
        pltpu.VMEM((2, _CC), _i32),
        pltpu.SemaphoreType.DMA,
        pltpu.SemaphoreType.DMA,
    ],
)
def _sc_pass_c(hsP, eaP, src_e, dst_e, rdenT, exT_in, aggT_out, bT_out,
               hs_v, agg_v, blo_v, bhi_v, rden_v,
               src0, dst0, ex0, ea0, src1, dst1, ex1, ea1, sem0, sem1):
    c = lax.axis_index("c")
    s = lax.axis_index("s")
    f0 = 64 * c + 4 * s
    h = lax.div(16 * c + s, 4)
    wq0 = 2 * lax.rem(s, 4)
    bq0 = 16 * h + wq0

    pltpu.sync_copy(hsP.at[pl.ds(lax.div(f0, 2), 2)], hs_v)
    pltpu.sync_copy(rdenT.at[h], rden_v)

    def zero(i, carry):
        z = jnp.zeros((16,), _f32)
        agg_v[0, pl.ds(i * 16, 16)] = z
        agg_v[1, pl.ds(i * 16, 16)] = z
        agg_v[2, pl.ds(i * 16, 16)] = z
        agg_v[3, pl.ds(i * 16, 16)] = z
        blo_v[0, pl.ds(i * 16, 16)] = z
        blo_v[1, pl.ds(i * 16, 16)] = z
        bhi_v[0, pl.ds(i * 16, 16)] = z
        bhi_v[1, pl.ds(i * 16, 16)] = z
        return carry
    lax.fori_loop(0, _N // 16, zero, 0)

    jv = [jnp.full((16,), j, _i32) for j in range(4)]
    jw = [jnp.full((16,), j, _i32) for j in range(2)]
    bufs = [(src0, dst0, ex0, ea0, sem0), (src1, dst1, ex1, ea1, sem1)]
    nch = _E // _CC

    def start(k, b):
        sv, dv, ev, eav, sem = bufs[b]
        e0 = k * _CC
        pltpu.async_copy(src_e.at[pl.ds(e0, _CC)], sv, sem)
        pltpu.async_copy(dst_e.at[pl.ds(e0, _CC)], dv, sem)
        pltpu.async_copy(exT_in.at[h, pl.ds(e0, _CC)], ev, sem)
        pltpu.async_copy(eaP.at[pl.ds(wq0, 2), pl.ds(e0, _CC)], eav, sem)

    def wait(b):
        sv, dv, ev, eav, sem = bufs[b]
        pltpu.make_async_copy(src_e.at[pl.ds(0, _CC)], sv, sem).wait()
        pltpu.make_async_copy(dst_e.at[pl.ds(0, _CC)], dv, sem).wait()
        pltpu.make_async_copy(exT_in.at[0, pl.ds(0, _CC)], ev, sem).wait()
        pltpu.make_async_copy(eaP.at[pl.ds(0, 2), pl.ds(0, _CC)], eav,
                              sem).wait()

    _GC = 5

    def process(b):
        sv, dv, ev, eav, _sem = bufs[b]

        def grp(g, carry2):
            dsts, exs, hsp, eas = [], [], [], []
            for u in range(_GC):
                sl = pl.ds((g * _GC + u) * 16, 16)
                srcv = sv[sl]
                dsts.append(dv[sl])
                exs.append(ev[sl])
                hsp.append([plsc.load_gather(hs_v, [jw[w], srcv])
                            for w in range(2)])
                eas.append([eav[w, sl] for w in range(2)])
            for u in range(_GC):
                exv = exs[u]
                for w in range(2):
                    g16 = hsp[u][w]
                    lo = plsc.bitcast(jnp.left_shift(g16, 16), _f32)
                    hi = plsc.bitcast(
                        jnp.bitwise_and(g16, jnp.int32(-65536)), _f32)
                    plsc.addupdate_scatter(agg_v, [jv[2 * w], dsts[u]],
                                           lo * exv)
                    plsc.addupdate_scatter(agg_v, [jv[2 * w + 1], dsts[u]],
                                           hi * exv)
                    e16 = eas[u][w]
                    elo = plsc.bitcast(jnp.left_shift(e16, 16), _f32)
                    ehi = plsc.bitcast(
                        jnp.bitwise_and(e16, jnp.int32(-65536)), _f32)
                    plsc.addupdate_scatter(blo_v, [jw[w], dsts[u]],
                                           elo * exv)
                    plsc.addupdate_scatter(bhi_v, [jw[w], dsts[u]],
                                           ehi * exv)
            return carry2
        lax.fori_loop(0, _CC // (16 * _GC), grp, 0)

    start(0, 0)
    start(1, 1)

    def body(i, carry):
        for b in (0, 1):
            k = 2 * i + b
            wait(b)
            process(b)

            @pl.when(k + 2 < nch)
            def _():
                start(k + 2, b)
        return carry
    lax.fori_loop(0, nch // 2, body, 0)

    def scale(i, carry):
        sl = pl.ds(i * 16, 16)
        rv = rden_v[sl]
        for r in range(4):
            agg_v[r, sl] = agg_v[r, sl] * rv
        for r in range(2):
            blo_v[r, sl] = blo_v[r, sl] * rv
            bhi_v[r, sl] = bhi_v[r, sl] * rv
        return carry
    lax.fori_loop(0, _N // 16, scale, 0)

    pltpu.sync_copy(agg_v, aggT_out.at[pl.ds(f0, 4)])
    pltpu.sync_copy(blo_v, bT_out.at[pl.ds(bq0, 2)])
    pltpu.sync_copy(bhi_v, bT_out.at[pl.ds(bq0 + 8, 2)])


def kernel(x, edge_index, edge_attr, W_src, W_dst, W_edge, a_src, a_dst,
           a_edge, W1, b1, W2, b2):
    v_src = (W_src.reshape(_D, _H, _DH) * a_src[None]).sum(-1)
    v_dst = (W_dst.reshape(_D, _H, _DH) * a_dst[None]).sum(-1)
    v_edge = (W_edge.reshape(_DE, _H, _DH) * a_edge[None]).sum(-1)
    w_src_p = jnp.concatenate([W_src[:, 0::2], W_src[:, 1::2]], axis=1)
    w_node = jnp.concatenate([w_src_p, v_src, v_dst], axis=1)
    w_bd = jax.scipy.linalg.block_diag(
        *[W_edge[:, 16 * hh:16 * hh + 16] for hh in range(_H)])

    nodeS, hsP = _node_pre(x, w_node)
    eaT = edge_attr.T
    seT, eaP = _edge_pre(eaT, v_edge.T)

    src_e = edge_index[0]
    dst_e = edge_index[1]
    exT, den_part = _sc_pass_a(nodeS, seT, src_e, dst_e)
    rdenT = _denred(den_part)
    aggT, bT = _sc_pass_c(hsP, eaP, src_e, dst_e, rdenT, exT)

    return _ffn(x, aggT, bT, w_bd, W1, b1, W2, b2)
